# Initial kernel scaffold; baseline (speedup 1.0000x reference)
import functools

import jax
import jax.numpy as jnp
from jax import lax
from jax.experimental import pallas as pl
from jax.experimental.pallas import tpu as pltpu
from jax.experimental.pallas import tpu_sc as plsc

NC = 2
NS = 16
NW = NC * NS

F32 = jnp.float32
_PREC = jax.lax.Precision.HIGHEST


@functools.partial(jax.jit, static_argnames=("n_ch", "ch"))
def _sc_seg_sum(h, src_t, dst_t, zeros_nf, *, n_ch, ch):
    n, f = h.shape
    rows_per_tile = n // NS
    mesh = plsc.VectorSubcoreMesh(core_axis_name="c", subcore_axis_name="s")

    @functools.partial(
        pl.kernel,
        out_type=jax.ShapeDtypeStruct((NC, n, f), F32),
        mesh=mesh,
        scratch_types=[
            pltpu.VMEM((n_ch, ch), jnp.int32),
            pltpu.VMEM((n_ch, ch), jnp.int32),
            pltpu.VMEM((2, ch, f), F32),
            pltpu.VMEM_SHARED((n, f), F32),
            pltpu.SemaphoreType.DMA((2,)),
            pltpu.SemaphoreType.DMA((2,)),
        ],
    )
    def k(h_hbm, src_hbm, dst_hbm, zero_hbm, out_hbm, sidx, didx, rows, acc,
          gsem, ssem):
        c = lax.axis_index("c")
        s = lax.axis_index("s")
        wid = s * NC + c
        r0 = s * rows_per_tile
        pltpu.sync_copy(zero_hbm.at[pl.ds(r0, rows_per_tile)],
                        acc.at[pl.ds(r0, rows_per_tile)])
        pltpu.sync_copy(src_hbm.at[wid], sidx)
        pltpu.sync_copy(dst_hbm.at[wid], didx)
        plsc.subcore_barrier()

        def gather_start(i, slot):
            pltpu.async_copy(h_hbm.at[sidx.at[i]], rows.at[slot],
                             gsem.at[slot])

        def gather_wait(i, slot):
            pltpu.make_async_copy(h_hbm.at[sidx.at[i]], rows.at[slot],
                                  gsem.at[slot]).wait()

        def scat_start(i, slot):
            pltpu.async_copy(rows.at[slot], acc.at[didx.at[i]],
                             ssem.at[slot], add=True)

        def scat_wait(i, slot):
            pltpu.make_async_copy(rows.at[slot], acc.at[didx.at[i]],
                                  ssem.at[slot]).wait()

        gather_start(0, 0)

        def body(i, _):
            slot = lax.rem(i, 2)
            nslot = lax.rem(i + 1, 2)
            gather_wait(i, slot)
            scat_start(i, slot)

            @pl.when(i >= 1)
            def _():
                scat_wait(i - 1, nslot)

            @pl.when(i + 1 < n_ch)
            def _():
                gather_start(i + 1, nslot)

            return 0

        lax.fori_loop(0, n_ch, body, 0, unroll=False)
        scat_wait(n_ch - 1, (n_ch - 1) % 2)
        plsc.subcore_barrier()
        pltpu.sync_copy(acc.at[pl.ds(r0, rows_per_tile)],
                        out_hbm.at[c, pl.ds(r0, rows_per_tile)])

    return k(h, src_t, dst_t, zeros_nf)


def _tc_mlp_layer(h, agg, batch2d, w1s, b1, w2bd, b2, scale, *, bn):
    n, f_in = h.shape
    f_out = w2bd.shape[1]
    b = 128
    grid = (n // bn,)

    def body(h_ref, agg_ref, bt_ref, w1_ref, b1_ref, w2_ref, b2_ref, sc_ref,
             hout_ref, pool_ref):
        i = pl.program_id(0)
        a = agg_ref[0] + agg_ref[1]
        if f_in == f_out:
            pre = h_ref[...] * sc_ref[...] + a
            z = jnp.dot(pre, w1_ref[...], preferred_element_type=F32,
                        precision=_PREC)
        else:
            xw = jnp.dot(h_ref[...], w1_ref[...], preferred_element_type=F32,
                         precision=_PREC)
            aw = jnp.dot(a, w1_ref[...], preferred_element_type=F32,
                         precision=_PREC)
            z = xw * sc_ref[...] + aw
        z = jnp.maximum(z + b1_ref[...], 0.0)
        z = jnp.dot(z, w2_ref[...], preferred_element_type=F32,
                    precision=_PREC) + b2_ref[...]
        hout_ref[...] = z
        onehot = (bt_ref[...] == lax.broadcasted_iota(jnp.int32, (bn, b), 1)
                  ).astype(F32)
        contrib = lax.dot_general(onehot, z, (((0,), (0,)), ((), ())),
                                  preferred_element_type=F32,
                                  precision=_PREC)

        @pl.when(i == 0)
        def _():
            pool_ref[...] = jnp.zeros_like(pool_ref)

        pool_ref[...] += contrib

    return pl.pallas_call(
        body,
        grid=grid,
        in_specs=[
            pl.BlockSpec((bn, f_in), lambda i: (i, 0)),
            pl.BlockSpec((2, bn, f_out), lambda i: (0, i, 0)),
            pl.BlockSpec((bn, 1), lambda i: (i, 0)),
            pl.BlockSpec((f_in, f_out), lambda i: (0, 0)),
            pl.BlockSpec((1, f_out), lambda i: (0, 0)),
            pl.BlockSpec((f_out, f_out), lambda i: (0, 0)),
            pl.BlockSpec((1, f_out), lambda i: (0, 0)),
            pl.BlockSpec((1, f_out), lambda i: (0, 0)),
        ],
        out_specs=[
            pl.BlockSpec((bn, f_out), lambda i: (i, 0)),
            pl.BlockSpec((b, f_out), lambda i: (0, 0)),
        ],
        out_shape=[
            jax.ShapeDtypeStruct((n, f_out), F32),
            jax.ShapeDtypeStruct((b, f_out), F32),
        ],
    )(h, agg, batch2d, w1s, b1, w2bd, b2, scale)


def _tc_assemble(pools, center2d, *, t_enc, hid, n_lay):
    b = pools.shape[1]
    width = t_enc * hid * n_lay

    def body(p_ref, c_ref, o_ref):
        for t in range(t_enc):
            for l in range(n_lay):
                blk = p_ref[l, :, t * hid:(t + 1) * hid]
                if t == 0:
                    blk = blk + c_ref[:, l * hid:(l + 1) * hid]
                o_ref[:, t * hid * n_lay + l * hid:
                      t * hid * n_lay + (l + 1) * hid] = blk

    return pl.pallas_call(
        body,
        out_shape=jax.ShapeDtypeStruct((b, width), F32),
    )(pools, center2d)


def kernel(x, edge_index, batch, params, center):
    n, d = x.shape
    e = edge_index.shape[1]
    t_enc = len(params)
    n_lay = len(params[0])
    hid = params[0][0]["W2"].shape[1]
    fe = t_enc * hid
    nb = 128

    ch = 80
    e_per_w = e // NW
    n_ch = e_per_w // ch
    assert e_per_w * NW == e and n_ch * ch == e_per_w
    src_t = edge_index[0].reshape(NW, n_ch, ch)
    dst_t = edge_index[1].reshape(NW, n_ch, ch)
    batch2d = batch.reshape(n, 1)
    zeros_d = jnp.zeros((n, d), F32)
    zeros_fe = jnp.zeros((n, fe), F32)

    w1_cat0 = jnp.concatenate([params[t][0]["W1"] for t in range(t_enc)], 1)
    w1_bd = [jax.scipy.linalg.block_diag(*[params[t][l]["W1"]
                                           for t in range(t_enc)])
             for l in range(1, n_lay)]
    w2_bd = [jax.scipy.linalg.block_diag(*[params[t][l]["W2"]
                                           for t in range(t_enc)])
             for l in range(n_lay)]
    b1 = [jnp.concatenate([params[t][l]["b1"] for t in range(t_enc)])
          .reshape(1, fe) for l in range(n_lay)]
    b2 = [jnp.concatenate([params[t][l]["b2"] for t in range(t_enc)])
          .reshape(1, fe) for l in range(n_lay)]
    scale = [jnp.repeat(jnp.stack([1.0 + params[t][l]["eps"]
                                   for t in range(t_enc)]), hid)
             .reshape(1, fe) for l in range(n_lay)]

    bn = 1000
    agg0 = _sc_seg_sum(x, src_t, dst_t, zeros_d, n_ch=n_ch, ch=ch)
    h, pool0 = _tc_mlp_layer(x, agg0, batch2d, w1_cat0, b1[0], w2_bd[0],
                             b2[0], scale[0], bn=bn)
    pools = [pool0]
    for l in range(1, n_lay):
        agg = _sc_seg_sum(h, src_t, dst_t, zeros_fe, n_ch=n_ch, ch=ch)
        h, pool_l = _tc_mlp_layer(h, agg, batch2d, w1_bd[l - 1], b1[l],
                                  w2_bd[l], b2[l], scale[l], bn=bn)
        pools.append(pool_l)

    z2d = _tc_assemble(jnp.stack(pools), center.reshape(1, hid * n_lay),
                       t_enc=t_enc, hid=hid, n_lay=n_lay)
    z_cat = z2d.reshape(nb, t_enc, hid * n_lay)
    return (z_cat, center)


# speedup vs baseline: 12.1475x; 12.1475x over previous
import functools

import jax
import jax.numpy as jnp
from jax import lax
from jax.experimental import pallas as pl
from jax.experimental.pallas import tpu as pltpu
from jax.experimental.pallas import tpu_sc as plsc

NC = 2
NS = 16

F32 = jnp.float32
_PREC = jax.lax.Precision.HIGHEST


@functools.partial(jax.jit, static_argnames=("n_ch", "ch"))
def _sc_seg_sum(h2, src_t, dst_t, zeros_nf, *, n_ch, ch):
    _, n, fh = h2.shape
    n_pad = zeros_nf.shape[0]
    rows_per_tile = n_pad // NS
    mesh = plsc.VectorSubcoreMesh(core_axis_name="c", subcore_axis_name="s")

    @functools.partial(
        pl.kernel,
        out_type=jax.ShapeDtypeStruct((NC, n_pad, fh), F32),
        mesh=mesh,
        scratch_types=[
            pltpu.VMEM((n_ch, ch), jnp.int32),
            pltpu.VMEM((n_ch, ch), jnp.int32),
            pltpu.VMEM((2, ch, fh), F32),
            pltpu.VMEM_SHARED((n_pad, fh), F32),
            pltpu.SemaphoreType.DMA((2,)),
            pltpu.SemaphoreType.DMA((2,)),
        ],
        compiler_params=pltpu.CompilerParams(use_tc_tiling_on_sc=False),
    )
    def k(h_hbm, src_hbm, dst_hbm, zero_hbm, out_hbm, sidx, didx, rows, acc,
          gsem, ssem):
        c = lax.axis_index("c")
        s = lax.axis_index("s")
        hv = h_hbm.at[c]
        r0 = s * rows_per_tile
        pltpu.sync_copy(zero_hbm.at[pl.ds(r0, rows_per_tile)],
                        acc.at[pl.ds(r0, rows_per_tile)])
        pltpu.sync_copy(src_hbm.at[s], sidx)
        pltpu.sync_copy(dst_hbm.at[s], didx)
        plsc.subcore_barrier()

        def gather_start(i, slot):
            pltpu.async_copy(hv.at[sidx.at[i]], rows.at[slot], gsem.at[slot])

        def gather_wait(i, slot):
            pltpu.make_async_copy(hv.at[sidx.at[i]], rows.at[slot],
                                  gsem.at[slot]).wait()

        def scat_start(i, slot):
            pltpu.async_copy(rows.at[slot], acc.at[didx.at[i]],
                             ssem.at[slot], add=True)

        def scat_wait(i, slot):
            pltpu.make_async_copy(rows.at[slot], acc.at[didx.at[i]],
                                  ssem.at[slot]).wait()

        gather_start(0, 0)

        def body(i, _):
            slot = lax.rem(i, 2)
            nslot = lax.rem(i + 1, 2)
            gather_wait(i, slot)
            scat_start(i, slot)

            @pl.when(i >= 1)
            def _():
                scat_wait(i - 1, nslot)

            @pl.when(i + 1 < n_ch)
            def _():
                gather_start(i + 1, nslot)

            return 0

        lax.fori_loop(0, n_ch, body, 0, unroll=False)
        scat_wait(n_ch - 1, (n_ch - 1) % 2)
        plsc.subcore_barrier()
        pltpu.sync_copy(acc.at[pl.ds(r0, rows_per_tile)],
                        out_hbm.at[c, pl.ds(r0, rows_per_tile)])

    return k(h2, src_t, dst_t, zeros_nf)


def _tc_mlp_layer(h2, agg, batch2d, w1s, b1, w2bd, b2, scale, *, bn):
    _, n, fh_in = h2.shape
    f_in = 2 * fh_in
    f_out = w2bd.shape[1]
    fh_out = f_out // 2
    b = 128
    grid = (n // bn,)

    def body(h_ref, agg_ref, bt_ref, w1_ref, b1_ref, w2_ref, b2_ref, sc_ref,
             hout_ref, pool_ref):
        i = pl.program_id(0)
        a = jnp.concatenate([agg_ref[0], agg_ref[1]], axis=-1)
        hcat = jnp.concatenate([h_ref[0], h_ref[1]], axis=-1)
        if f_in == f_out:
            pre = hcat * sc_ref[...] + a
            z = jnp.dot(pre, w1_ref[...], preferred_element_type=F32,
                        precision=_PREC)
        else:
            xw = jnp.dot(hcat, w1_ref[...], preferred_element_type=F32,
                         precision=_PREC)
            aw = jnp.dot(a, w1_ref[...], preferred_element_type=F32,
                         precision=_PREC)
            z = xw * sc_ref[...] + aw
        z = jnp.maximum(z + b1_ref[...], 0.0)
        z = jnp.dot(z, w2_ref[...], preferred_element_type=F32,
                    precision=_PREC) + b2_ref[...]
        hout_ref[0] = z[:, :fh_out]
        hout_ref[1] = z[:, fh_out:]
        onehot = (bt_ref[...] == lax.broadcasted_iota(jnp.int32, (bn, b), 1)
                  ).astype(F32)
        contrib = lax.dot_general(onehot, z, (((0,), (0,)), ((), ())),
                                  preferred_element_type=F32,
                                  precision=_PREC)

        @pl.when(i == 0)
        def _():
            pool_ref[...] = jnp.zeros_like(pool_ref)

        pool_ref[...] += contrib

    return pl.pallas_call(
        body,
        grid=grid,
        in_specs=[
            pl.BlockSpec((2, bn, fh_in), lambda i: (0, i, 0)),
            pl.BlockSpec((2, bn, fh_in), lambda i: (0, i, 0)),
            pl.BlockSpec((bn, 1), lambda i: (i, 0)),
            pl.BlockSpec((f_in, f_out), lambda i: (0, 0)),
            pl.BlockSpec((1, f_out), lambda i: (0, 0)),
            pl.BlockSpec((f_out, f_out), lambda i: (0, 0)),
            pl.BlockSpec((1, f_out), lambda i: (0, 0)),
            pl.BlockSpec((1, f_out), lambda i: (0, 0)),
        ],
        out_specs=[
            pl.BlockSpec((2, bn, fh_out), lambda i: (0, i, 0)),
            pl.BlockSpec((b, f_out), lambda i: (0, 0)),
        ],
        out_shape=[
            jax.ShapeDtypeStruct((2, n, fh_out), F32),
            jax.ShapeDtypeStruct((b, f_out), F32),
        ],
    )(h2, agg, batch2d, w1s, b1, w2bd, b2, scale)


def _tc_assemble(pools, center2d, *, t_enc, hid, n_lay):
    b = pools.shape[1]
    width = t_enc * hid * n_lay

    def body(p_ref, c_ref, o_ref):
        for t in range(t_enc):
            for l in range(n_lay):
                blk = p_ref[l, :, t * hid:(t + 1) * hid]
                if t == 0:
                    blk = blk + c_ref[:, l * hid:(l + 1) * hid]
                o_ref[:, t * hid * n_lay + l * hid:
                      t * hid * n_lay + (l + 1) * hid] = blk

    return pl.pallas_call(
        body,
        out_shape=jax.ShapeDtypeStruct((b, width), F32),
    )(pools, center2d)


def kernel(x, edge_index, batch, params, center):
    n, d = x.shape
    e = edge_index.shape[1]
    t_enc = len(params)
    n_lay = len(params[0])
    hid = params[0][0]["W2"].shape[1]
    fe = t_enc * hid
    nb = 128

    ch = 80
    e_per_t = e // NS
    n_ch = e_per_t // ch
    assert e_per_t * NS == e and n_ch * ch == e_per_t
    src_t = edge_index[0].reshape(NS, n_ch, ch)
    dst_t = edge_index[1].reshape(NS, n_ch, ch)
    batch2d = batch.reshape(n, 1)
    n_pad = -(-n // (8 * NS)) * (8 * NS)
    zeros_d = jnp.zeros((n_pad, d // 2), F32)
    zeros_fe = jnp.zeros((n_pad, fe // 2), F32)
    x2 = jnp.stack([x[:, :d // 2], x[:, d // 2:]])

    w1_cat0 = jnp.concatenate([params[t][0]["W1"] for t in range(t_enc)], 1)
    w1_bd = [jax.scipy.linalg.block_diag(*[params[t][l]["W1"]
                                           for t in range(t_enc)])
             for l in range(1, n_lay)]
    w2_bd = [jax.scipy.linalg.block_diag(*[params[t][l]["W2"]
                                           for t in range(t_enc)])
             for l in range(n_lay)]
    b1 = [jnp.concatenate([params[t][l]["b1"] for t in range(t_enc)])
          .reshape(1, fe) for l in range(n_lay)]
    b2 = [jnp.concatenate([params[t][l]["b2"] for t in range(t_enc)])
          .reshape(1, fe) for l in range(n_lay)]
    scale = [jnp.repeat(jnp.stack([1.0 + params[t][l]["eps"]
                                   for t in range(t_enc)]), hid)
             .reshape(1, fe) for l in range(n_lay)]

    bn = 1000
    agg0 = _sc_seg_sum(x2, src_t, dst_t, zeros_d, n_ch=n_ch, ch=ch)
    h2, pool0 = _tc_mlp_layer(x2, agg0, batch2d, w1_cat0, b1[0], w2_bd[0],
                              b2[0], scale[0], bn=bn)
    pools = [pool0]
    for l in range(1, n_lay):
        agg = _sc_seg_sum(h2, src_t, dst_t, zeros_fe, n_ch=n_ch, ch=ch)
        h2, pool_l = _tc_mlp_layer(h2, agg, batch2d, w1_bd[l - 1], b1[l],
                                   w2_bd[l], b2[l], scale[l], bn=bn)
        pools.append(pool_l)

    z2d = _tc_assemble(jnp.stack(pools), center.reshape(1, hid * n_lay),
                       t_enc=t_enc, hid=hid, n_lay=n_lay)
    z_cat = z2d.reshape(nb, t_enc, hid * n_lay)
    return (z_cat, center)
